# baseline (speedup 1.0000x reference)
import functools

import jax
import jax.numpy as jnp
from jax import lax
from jax.experimental import pallas as pl
from jax.experimental.pallas import tpu as pltpu
from jax.experimental.pallas import tpu_sc as plsc

N = 100000
D = 128
NFEAT = 17

_BN = 12800
PARTS = (12800, 38400, 48800)
P_OFF = (0, 12800, 51200)

NC = 2
NS = 16
NW = NC * NS

C = 400


def _make_sc_gather(base_row, nrows):
    nchunk = nrows // C
    q, r = divmod(nchunk, NW)
    tmax = q + (1 if r else 0)
    mesh = plsc.VectorSubcoreMesh(core_axis_name="c", subcore_axis_name="s")

    @functools.partial(
        pl.kernel,
        mesh=mesh,
        out_type=jax.ShapeDtypeStruct((nrows, D), jnp.float32),
        scratch_types=[
            pltpu.VMEM((tmax * C,), jnp.int32),
            pltpu.VMEM((C, D), jnp.float32),
            pltpu.VMEM((C, D), jnp.float32),
            pltpu.SemaphoreType.DMA,
            pltpu.SemaphoreType.DMA,
            pltpu.SemaphoreType.DMA,
        ],
    )
    def sc_gather(idx_hbm, table_hbm, out_hbm,
                  idxv, emb0, emb1,
                  gsem0, gsem1, isem):
        wid = lax.axis_index("s") * NC + lax.axis_index("c")
        mycount = q + jnp.where(wid < r, 1, 0) if r else q
        base_chunk = wid * q + jnp.minimum(wid, r)
        row0 = base_row + base_chunk * C
        embs = (emb0, emb1)
        gsems = (gsem0, gsem1)

        if q > 0:
            pltpu.sync_copy(
                idx_hbm.at[pl.ds(row0, q * C)], idxv.at[pl.ds(0, q * C)])
        if r:
            @pl.when(wid < r)
            def _():
                pltpu.async_copy(
                    idx_hbm.at[pl.ds(row0 + q * C, C)],
                    idxv.at[pl.ds(q * C, C)], isem).wait()

        def start(t):
            s = t % 2

            @pl.when(t < mycount)
            def _():
                pltpu.async_copy(
                    table_hbm.at[idxv.at[pl.ds(t * C, C)]], embs[s], gsems[s])

        def finish(t):
            s = t % 2

            @pl.when(t < mycount)
            def _():
                pltpu.make_async_copy(
                    table_hbm.at[idxv.at[pl.ds(t * C, C)]],
                    embs[s], gsems[s]).wait()
                pltpu.async_copy(
                    embs[s],
                    out_hbm.at[pl.ds((base_chunk + t) * C, C)], gsems[s])

        def drain(t):
            s = t % 2

            @pl.when(t < mycount)
            def _():
                pltpu.make_async_copy(
                    embs[s],
                    out_hbm.at[pl.ds((base_chunk + t) * C, C)],
                    gsems[s]).wait()

        start(0)
        for t in range(tmax):
            if t + 1 < tmax:
                if t - 1 >= 0:
                    drain(t - 1)
                start(t + 1)
            finish(t)
        for t in range(max(tmax - 2, 0), tmax):
            drain(t)

    return sc_gather


def _tc_body_first(xt_ref, g_ref, w_ref, b_ref, o_ref):
    xf = xt_ref[...].astype(jnp.float32)
    mm = lax.dot_general(
        xf, w_ref[...], (((0,), (0,)), ((), ())),
        preferred_element_type=jnp.float32,
    )
    o_ref[...] = mm + b_ref[...] + g_ref[...]


def _tc_body_chain(part_ref, xt_ref, g_ref, w_ref, b_ref, o_ref):
    del part_ref
    _tc_body_first(xt_ref, g_ref, w_ref, b_ref, o_ref)


def _make_tc(part_idx):
    nrows = PARTS[part_idx]
    boff = P_OFF[part_idx] // _BN
    nb = (nrows + _BN - 1) // _BN
    specs = [
        pl.BlockSpec((NFEAT, _BN), lambda i: (0, i + boff)),
        pl.BlockSpec((_BN, D), lambda i: (i, 0)),
        pl.BlockSpec((NFEAT, D), lambda i: (0, 0)),
        pl.BlockSpec((1, D), lambda i: (0, 0)),
    ]
    if part_idx == 0:
        return pl.pallas_call(
            _tc_body_first,
            grid=(nb,),
            in_specs=specs,
            out_specs=pl.BlockSpec((_BN, D), lambda i: (i + boff, 0)),
            out_shape=jax.ShapeDtypeStruct((N, D), jnp.float32),
        )
    return pl.pallas_call(
        _tc_body_chain,
        grid=(nb,),
        in_specs=[pl.BlockSpec(memory_space=pltpu.HBM)] + specs,
        out_specs=pl.BlockSpec((_BN, D), lambda i: (i + boff, 0)),
        out_shape=jax.ShapeDtypeStruct((N, D), jnp.float32),
        input_output_aliases={0: 0},
    )


_tc_calls = tuple(_make_tc(k) for k in range(len(PARTS)))
_sc_calls = tuple(
    _make_sc_gather(P_OFF[k], PARTS[k]) for k in range(len(PARTS)))


def kernel(x, type_embed, W, b):
    xt = x.T
    idx = xt[0]
    Wp = jnp.concatenate([jnp.zeros((1, D), W.dtype), W], axis=0)
    b2 = b.reshape(1, D)
    gs = [sc(idx, type_embed) for sc in _sc_calls]
    out = _tc_calls[0](xt, gs[0], Wp, b2)
    for k in range(1, len(PARTS)):
        out = _tc_calls[k](out, xt, gs[k], Wp, b2)
    return out


# speedup vs baseline: 1.0427x; 1.0427x over previous
import functools

import jax
import jax.numpy as jnp
from jax import lax
from jax.experimental import pallas as pl
from jax.experimental.pallas import tpu as pltpu
from jax.experimental.pallas import tpu_sc as plsc

N = 100000
D = 128
NFEAT = 17

_BN = 12800
PARTS = (51200, 48800)
P_OFF = (0, 51200)

NC = 2
NS = 16
NW = NC * NS

C = 200
NBUF = 3


def _make_sc_gather(base_row, nrows):
    nchunk = nrows // C
    q, r = divmod(nchunk, NW)
    tmax = q + (1 if r else 0)
    mesh = plsc.VectorSubcoreMesh(core_axis_name="c", subcore_axis_name="s")

    @functools.partial(
        pl.kernel,
        mesh=mesh,
        out_type=jax.ShapeDtypeStruct((nrows, D), jnp.float32),
        scratch_types=[
            pltpu.VMEM((tmax * C,), jnp.int32),
            pltpu.VMEM((C, D), jnp.float32),
            pltpu.VMEM((C, D), jnp.float32),
            pltpu.VMEM((C, D), jnp.float32),
            pltpu.SemaphoreType.DMA,
            pltpu.SemaphoreType.DMA,
            pltpu.SemaphoreType.DMA,
            pltpu.SemaphoreType.DMA,
        ],
    )
    def sc_gather(idx_hbm, table_hbm, out_hbm,
                  idxv, emb0, emb1, emb2,
                  gsem0, gsem1, gsem2, isem):
        wid = lax.axis_index("s") * NC + lax.axis_index("c")
        mycount = q + jnp.where(wid < r, 1, 0) if r else q
        base_chunk = wid * q + jnp.minimum(wid, r)
        row0 = base_row + base_chunk * C
        embs = (emb0, emb1, emb2)
        gsems = (gsem0, gsem1, gsem2)

        if q > 0:
            pltpu.sync_copy(
                idx_hbm.at[pl.ds(row0, q * C)], idxv.at[pl.ds(0, q * C)])
        if r:
            @pl.when(wid < r)
            def _():
                pltpu.async_copy(
                    idx_hbm.at[pl.ds(row0 + q * C, C)],
                    idxv.at[pl.ds(q * C, C)], isem).wait()

        def start(t):
            s = t % NBUF

            @pl.when(t < mycount)
            def _():
                pltpu.async_copy(
                    table_hbm.at[idxv.at[pl.ds(t * C, C)]], embs[s], gsems[s])

        def finish(t):
            s = t % NBUF

            @pl.when(t < mycount)
            def _():
                pltpu.make_async_copy(
                    table_hbm.at[idxv.at[pl.ds(t * C, C)]],
                    embs[s], gsems[s]).wait()
                pltpu.async_copy(
                    embs[s],
                    out_hbm.at[pl.ds((base_chunk + t) * C, C)], gsems[s])

        def drain(t):
            s = t % NBUF

            @pl.when(t < mycount)
            def _():
                pltpu.make_async_copy(
                    embs[s],
                    out_hbm.at[pl.ds((base_chunk + t) * C, C)],
                    gsems[s]).wait()

        for u in range(min(NBUF - 1, tmax)):
            start(u)
        for t in range(tmax):
            if t + NBUF - 1 < tmax:
                if t - 1 >= 0:
                    drain(t - 1)
                start(t + NBUF - 1)
            finish(t)
        for t in range(max(tmax - NBUF, 0), tmax):
            drain(t)

    return sc_gather


def _tc_body_first(xt_ref, g_ref, w_ref, b_ref, o_ref):
    xf = xt_ref[...].astype(jnp.float32)
    mm = lax.dot_general(
        xf, w_ref[...], (((0,), (0,)), ((), ())),
        preferred_element_type=jnp.float32,
    )
    o_ref[...] = mm + b_ref[...] + g_ref[...]


def _tc_body_chain(part_ref, xt_ref, g_ref, w_ref, b_ref, o_ref):
    del part_ref
    _tc_body_first(xt_ref, g_ref, w_ref, b_ref, o_ref)


def _make_tc(part_idx):
    nrows = PARTS[part_idx]
    boff = P_OFF[part_idx] // _BN
    nb = (nrows + _BN - 1) // _BN
    specs = [
        pl.BlockSpec((NFEAT, _BN), lambda i: (0, i + boff)),
        pl.BlockSpec((_BN, D), lambda i: (i, 0)),
        pl.BlockSpec((NFEAT, D), lambda i: (0, 0)),
        pl.BlockSpec((1, D), lambda i: (0, 0)),
    ]
    if part_idx == 0:
        return pl.pallas_call(
            _tc_body_first,
            grid=(nb,),
            in_specs=specs,
            out_specs=pl.BlockSpec((_BN, D), lambda i: (i + boff, 0)),
            out_shape=jax.ShapeDtypeStruct((N, D), jnp.float32),
        )
    return pl.pallas_call(
        _tc_body_chain,
        grid=(nb,),
        in_specs=[pl.BlockSpec(memory_space=pltpu.HBM)] + specs,
        out_specs=pl.BlockSpec((_BN, D), lambda i: (i + boff, 0)),
        out_shape=jax.ShapeDtypeStruct((N, D), jnp.float32),
        input_output_aliases={0: 0},
    )


_tc_calls = tuple(_make_tc(k) for k in range(len(PARTS)))
_sc_calls = tuple(
    _make_sc_gather(P_OFF[k], PARTS[k]) for k in range(len(PARTS)))


def kernel(x, type_embed, W, b):
    xt = x.T
    idx = xt[0]
    Wp = jnp.concatenate([jnp.zeros((1, D), W.dtype), W], axis=0)
    b2 = b.reshape(1, D)
    gs = [sc(idx, type_embed) for sc in _sc_calls]
    out = _tc_calls[0](xt, gs[0], Wp, b2)
    for k in range(1, len(PARTS)):
        out = _tc_calls[k](out, xt, gs[k], Wp, b2)
    return out
